# baseline (speedup 1.0000x reference)
import jax
import jax.numpy as jnp
from jax.experimental import pallas as pl

_DELTA = 0.05
_S = 128 * 128
_K = int(_S * 0.3 - 1) + 1
_HI0 = 0x40000000


def _region_loss_kernel(t_ref, p_ref, out_ref):
    t = t_ref[...]
    p = p_ref[...]
    t0 = t[:, :_S]
    t1 = t[:, _S:2 * _S]
    t2 = t[:, 2 * _S:]
    gray = 0.39 * t0 + 0.5 * t1 + 0.11 * t2
    gi = jax.lax.bitcast_convert_type(gray, jnp.int32)
    b = gray.shape[0]

    ones_s = jnp.ones((_S, 8), jnp.float32)

    def body(_, carry):
        lo, hi = carry
        mid = lo + ((hi - lo) >> 1)
        mask = jnp.where(gi >= mid, jnp.float32(1), jnp.float32(0))
        cnt = jax.lax.dot_general(
            mask, ones_s, (((1,), (0,)), ((), ())),
            preferred_element_type=jnp.float32)[:, :1]
        ge = cnt >= _K
        return jnp.where(ge, mid, lo), jnp.where(ge, hi, mid)

    lo0 = jnp.zeros((b, 1), jnp.int32)
    hi0 = jnp.full((b, 1), _HI0, jnp.int32)
    lo, _ = jax.lax.fori_loop(0, 30, body, (lo0, hi0))
    yu = jax.lax.bitcast_convert_type(lo, jnp.float32)

    d = jnp.abs(t - p)
    f = jnp.where(d < _DELTA, 0.5 * d * d, _DELTA * d - 0.5 * _DELTA * _DELTA)
    m = (gray >= yu).astype(jnp.float32)
    fm = (f[:, :_S] + f[:, _S:2 * _S] + f[:, 2 * _S:]) * m
    ones3s = jnp.ones((3 * _S, 8), jnp.float32)
    sf = jax.lax.dot_general(f, ones3s, (((1,), (0,)), ((), ())),
                             preferred_element_type=jnp.float32)[:, :1]
    sfm = jax.lax.dot_general(fm, ones3s[:_S], (((1,), (0,)), ((), ())),
                              preferred_element_type=jnp.float32)[:, :1]
    out_ref[...] = jnp.sum(sf + 3.0 * sfm).reshape(1, 1, 1)


def kernel(pred, target):
    n, c, h, w = pred.shape
    s = h * w
    pr = pred.reshape(n, c * s)
    tr = target.reshape(n, c * s)
    blk = 32
    grid = n // blk
    partial = pl.pallas_call(
        _region_loss_kernel,
        grid=(grid,),
        in_specs=[
            pl.BlockSpec((blk, c * s), lambda i: (i, 0)),
            pl.BlockSpec((blk, c * s), lambda i: (i, 0)),
        ],
        out_specs=pl.BlockSpec((1, 1, 1), lambda i: (i, 0, 0)),
        out_shape=jax.ShapeDtypeStruct((grid, 1, 1), jnp.float32),
    )(tr, pr)
    return jnp.sum(partial) * (1.0 / (n * c * s))


# speedup vs baseline: 1.3886x; 1.3886x over previous
import jax
import jax.numpy as jnp
from jax.experimental import pallas as pl

_DELTA = 0.05
_S = 128 * 128
_K = int(_S * 0.3 - 1) + 1
_HI0 = 0x40000000


def _region_loss_kernel(t_ref, p_ref, out_ref):
    t = t_ref[...]
    p = p_ref[...]
    t0 = t[:, :_S]
    t1 = t[:, _S:2 * _S]
    t2 = t[:, 2 * _S:]
    gray = 0.39 * t0 + 0.5 * t1 + 0.11 * t2
    gi = jax.lax.bitcast_convert_type(gray, jnp.int32)
    b = gray.shape[0]

    def body(_, carry):
        lo, hi = carry
        mid = lo + ((hi - lo) >> 1)
        cnt = jnp.sum((gi >= mid).astype(jnp.int32), axis=1, keepdims=True)
        ge = cnt >= _K
        return jnp.where(ge, mid, lo), jnp.where(ge, hi, mid)

    lo0 = jnp.zeros((b, 1), jnp.int32)
    hi0 = jnp.full((b, 1), _HI0, jnp.int32)
    lo, _ = jax.lax.fori_loop(0, 30, body, (lo0, hi0))
    yu = jax.lax.bitcast_convert_type(lo, jnp.float32)

    d = jnp.abs(t - p)
    f = jnp.where(d < _DELTA, 0.5 * d * d, _DELTA * d - 0.5 * _DELTA * _DELTA)
    m = (gray >= yu).astype(jnp.float32)
    fm = (f[:, :_S] + f[:, _S:2 * _S] + f[:, 2 * _S:]) * m
    ones3s = jnp.ones((3 * _S, 8), jnp.float32)
    sf = jax.lax.dot_general(f, ones3s, (((1,), (0,)), ((), ())),
                             preferred_element_type=jnp.float32)[:, :1]
    sfm = jax.lax.dot_general(fm, ones3s[:_S], (((1,), (0,)), ((), ())),
                              preferred_element_type=jnp.float32)[:, :1]
    out_ref[...] = jnp.sum(sf + 3.0 * sfm).reshape(1, 1, 1)


def kernel(pred, target):
    n, c, h, w = pred.shape
    s = h * w
    pr = pred.reshape(n, c * s)
    tr = target.reshape(n, c * s)
    blk = 32
    grid = n // blk
    partial = pl.pallas_call(
        _region_loss_kernel,
        grid=(grid,),
        in_specs=[
            pl.BlockSpec((blk, c * s), lambda i: (i, 0)),
            pl.BlockSpec((blk, c * s), lambda i: (i, 0)),
        ],
        out_specs=pl.BlockSpec((1, 1, 1), lambda i: (i, 0, 0)),
        out_shape=jax.ShapeDtypeStruct((grid, 1, 1), jnp.float32),
    )(tr, pr)
    return jnp.sum(partial) * (1.0 / (n * c * s))
